# baseline (speedup 1.0000x reference)
import functools

import jax
import jax.numpy as jnp
from jax import lax
from jax.experimental import pallas as pl
from jax.experimental.pallas import tpu as pltpu
from jax.experimental.pallas import tpu_sc as plsc

_NUM_BINS = 32
_NUM_FEATURES = 100
_BATCH = 16384
_NUM_PAIRS = 10

_NC = 2
_NS = 16
_NW = _NC * _NS

_TC_BLOCK = 1024
_TC_ROWS = 6144
_SC_ROWS = _BATCH - _TC_ROWS

_BPW = _SC_ROWS // _NW
_GROUPS = _BPW // 16


def _ebm_body(x_hbm, wm_hbm, wi_hbm, ic_hbm, out_hbm,
              x_v, wm_v, wi_v, ic_v, out_v):
    wid = lax.axis_index("s") * _NC + lax.axis_index("c")
    base = wid * _BPW
    pltpu.sync_copy(x_hbm.at[pl.ds(base * _NUM_FEATURES, _BPW * _NUM_FEATURES)],
                    x_v)
    pltpu.sync_copy(wm_hbm, wm_v)
    pltpu.sync_copy(wi_hbm, wi_v)
    pltpu.sync_copy(ic_hbm, ic_v)

    lane_row = lax.iota(jnp.int32, 16) * _NUM_FEATURES
    ones = jnp.ones((16,), jnp.int32)

    def group(g, carry):
        accs = [ic_v[...],
                jnp.zeros((16,), jnp.float32),
                jnp.zeros((16,), jnp.float32),
                jnp.zeros((16,), jnp.float32)]
        xidx = lane_row + g * (16 * _NUM_FEATURES)
        prev_bi = None
        for f in range(_NUM_FEATURES):
            xv = plsc.load_gather(x_v, [xidx])
            xidx = xidx + ones
            bi = (xv * float(_NUM_BINS)).astype(jnp.int32)
            accs[f % 4] = accs[f % 4] + plsc.load_gather(
                wm_v, [bi + f * _NUM_BINS])
            if f < 2 * _NUM_PAIRS:
                if f % 2 == 0:
                    prev_bi = bi
                else:
                    k = f // 2
                    flat = (prev_bi * _NUM_BINS + bi
                            + k * (_NUM_BINS * _NUM_BINS))
                    accs[(f + 1) % 4] = accs[(f + 1) % 4] + plsc.load_gather(
                        wi_v, [flat])
        acc = (accs[0] + accs[1]) + (accs[2] + accs[3])
        out_v[pl.ds(g * 16, 16)] = acc
        return carry

    lax.fori_loop(0, _GROUPS, group, 0)
    pltpu.sync_copy(out_v, out_hbm.at[pl.ds(base, _BPW)])


_ebm_sc = functools.partial(
    pl.kernel,
    out_type=jax.ShapeDtypeStruct((_SC_ROWS,), jnp.float32),
    mesh=plsc.VectorSubcoreMesh(core_axis_name="c", subcore_axis_name="s"),
    compiler_params=pltpu.CompilerParams(needs_layout_passes=False),
    scratch_types=[
        pltpu.VMEM((_BPW * _NUM_FEATURES,), jnp.float32),
        pltpu.VMEM((_NUM_FEATURES * _NUM_BINS,), jnp.float32),
        pltpu.VMEM((_NUM_PAIRS * _NUM_BINS * _NUM_BINS,), jnp.float32),
        pltpu.VMEM((16,), jnp.float32),
        pltpu.VMEM((_BPW,), jnp.float32),
    ],
)(_ebm_body)


def _tc_body(x_ref, wmt_ref, wi_ref, ic_ref, o_ref):
    x = x_ref[...]
    bi = (x * float(_NUM_BINS)).astype(jnp.int32)
    g = jnp.zeros(x.shape, jnp.float32)
    for j in range(_NUM_BINS):
        g = g + jnp.where(bi == j, wmt_ref[j, :][None, :], 0.0)
    preds = jnp.sum(g, axis=1) + ic_ref[0, 0]
    bins_iota = jax.lax.broadcasted_iota(jnp.int32, (1, _NUM_BINS), 1)
    for k in range(_NUM_PAIRS):
        ii = bi[:, 2 * k:2 * k + 1]
        jj = bi[:, 2 * k + 1:2 * k + 2]
        ohi = (ii == bins_iota).astype(jnp.float32)
        row = jnp.dot(ohi, wi_ref[k],
                      preferred_element_type=jnp.float32)
        preds = preds + jnp.sum(
            jnp.where(jj == bins_iota, row, 0.0), axis=1)
    o_ref[...] = preds[:, None]


_ebm_tc = pl.pallas_call(
    _tc_body,
    grid=(_TC_ROWS // _TC_BLOCK,),
    in_specs=[
        pl.BlockSpec((_TC_BLOCK, _NUM_FEATURES), lambda i: (i, 0)),
        pl.BlockSpec((_NUM_BINS, _NUM_FEATURES), lambda i: (0, 0)),
        pl.BlockSpec((_NUM_PAIRS, _NUM_BINS, _NUM_BINS), lambda i: (0, 0, 0)),
        pl.BlockSpec((1, 1), lambda i: (0, 0)),
    ],
    out_specs=pl.BlockSpec((_TC_BLOCK, 1), lambda i: (i, 0)),
    out_shape=jax.ShapeDtypeStruct((_TC_ROWS, 1), jnp.float32),
)


def kernel(inputs, W_main, W_inter, intercept):
    wm = W_main.reshape(-1)
    wi = W_inter.reshape(-1)
    ic = jnp.broadcast_to(intercept.astype(jnp.float32), (16,))
    x_sc = inputs[:_SC_ROWS].reshape(-1)
    sc_out = _ebm_sc(x_sc, wm, wi, ic)

    x_tc = inputs[_SC_ROWS:]
    tc_out = _ebm_tc(x_tc, W_main.T, W_inter,
                     intercept.reshape(1, 1).astype(jnp.float32))
    return jnp.concatenate([sc_out.reshape(-1, 1), tc_out], axis=0)
